# Initial kernel scaffold; baseline (speedup 1.0000x reference)
import jax
import jax.numpy as jnp
from jax.experimental import pallas as pl


def kernel(updates, mask):
    raise NotImplementedError("write your pallas kernel here")


# speedup vs baseline: 9.5594x; 9.5594x over previous
import functools

import jax
import jax.numpy as jnp
from jax import lax
from jax.experimental import pallas as pl
from jax.experimental.pallas import tpu as pltpu
from jax.experimental.pallas import tpu_sc as plsc

B, H, W, C = 4, 128, 128, 192
H2, W2 = 2 * H, 2 * W
N_IN = H * W * C
N_OUT = H2 * W2 * C

NC, NS, L = 2, 16, 16
WIN = 1_572_864
PASSES = N_OUT // WIN
TRASH = 8192
PER_TILE = N_IN // NS
CHUNK = 4096
N_CHUNKS = PER_TILE // CHUNK
ZCHUNK = 16384
TILE_WIN = WIN // NS
BATCHES_PER_CORE = B // NC


def _unpool_body(upd_hbm, mask_hbm, out_hbm, idx_v, val_v, lidx_v, zero_v,
                 win_sh):
    c = lax.axis_index("c")
    s = lax.axis_index("s")

    def zinit(j, carry):
        zero_v[pl.ds(j * L, L)] = jnp.zeros((L,), jnp.float32)
        return carry

    lax.fori_loop(0, ZCHUNK // L, zinit, 0)

    for bi in range(BATCHES_PER_CORE):
        b = bi * NC + c
        in_base = b * N_IN + s * PER_TILE
        for p in range(PASSES):
            lo = p * WIN
            hi = lo + WIN

            for z in range(TILE_WIN // ZCHUNK):
                pltpu.sync_copy(
                    zero_v,
                    win_sh.at[pl.ds(s * TILE_WIN + z * ZCHUNK, ZCHUNK)])
            plsc.subcore_barrier()

            def chunk_body(i, carry, in_base=in_base, lo=lo, hi=hi):
                start = in_base + i * CHUNK
                pltpu.sync_copy(mask_hbm.at[pl.ds(start, CHUNK)], idx_v)
                pltpu.sync_copy(upd_hbm.at[pl.ds(start, CHUNK)], val_v)

                def vec_body(j, carry2):
                    iv = idx_v[pl.ds(j * L, L)]
                    inwin = (iv >= lo) & (iv < hi)
                    lidx = jnp.where(inwin, iv - lo,
                                     WIN + (iv & (TRASH - 1)))
                    lidx_v[pl.ds(j * L, L)] = lidx
                    return carry2

                lax.fori_loop(0, CHUNK // L, vec_body, 0)
                pltpu.sync_copy(val_v, win_sh.at[lidx_v], add=True)
                return carry

            lax.fori_loop(0, N_CHUNKS, chunk_body, 0)
            plsc.subcore_barrier()

            out_start = b * N_OUT + lo + s * TILE_WIN
            pltpu.sync_copy(win_sh.at[pl.ds(s * TILE_WIN, TILE_WIN)],
                            out_hbm.at[pl.ds(out_start, TILE_WIN)])
            plsc.subcore_barrier()


_unpool = pl.kernel(
    _unpool_body,
    out_type=jax.ShapeDtypeStruct((B * N_OUT,), jnp.float32),
    mesh=plsc.VectorSubcoreMesh(core_axis_name="c", subcore_axis_name="s",
                                num_cores=NC, num_subcores=NS),
    scratch_types=[
        pltpu.VMEM((CHUNK,), jnp.int32),
        pltpu.VMEM((CHUNK,), jnp.float32),
        pltpu.VMEM((CHUNK,), jnp.int32),
        pltpu.VMEM((ZCHUNK,), jnp.float32),
        pltpu.VMEM_SHARED((WIN + TRASH,), jnp.float32),
    ],
)


@jax.jit
def kernel(updates, mask):
    upd = updates.reshape(-1)
    msk = mask.reshape(-1).astype(jnp.int32)
    out = _unpool(upd, msk)
    return out.reshape(B, H2, W2, C)


# speedup vs baseline: 13.4161x; 1.4034x over previous
import jax
import jax.numpy as jnp
from jax import lax
from jax.experimental import pallas as pl
from jax.experimental.pallas import tpu as pltpu
from jax.experimental.pallas import tpu_sc as plsc

B, H, W, C = 4, 128, 128, 192
H2, W2 = 2 * H, 2 * W
N_IN = H * W * C
N_OUT = H2 * W2 * C

NC, NS, L = 2, 16, 16
WIN = 1_572_864
PASSES = N_OUT // WIN
SPREAD = (1 << 20) - 1
PER_TILE = N_IN // NS
CHUNK = 3072
N_CHUNKS = PER_TILE // CHUNK
TILE_WIN = WIN // NS
BATCHES_PER_CORE = B // NC
NSETS = 4
N_GROUPS = N_CHUNKS // NSETS


def _unpool_body(upd_hbm, mask_hbm, out_hbm,
                 idx_v0, idx_v1, idx_v2, idx_v3,
                 val_v0, val_v1, val_v2, val_v3,
                 win_sh,
                 isem0, isem1, isem2, isem3,
                 vsem0, vsem1, vsem2, vsem3,
                 asem0, asem1, asem2, asem3):
    idx_v = (idx_v0, idx_v1, idx_v2, idx_v3)
    val_v = (val_v0, val_v1, val_v2, val_v3)
    isem = (isem0, isem1, isem2, isem3)
    vsem = (vsem0, vsem1, vsem2, vsem3)
    asem = (asem0, asem1, asem2, asem3)
    c = lax.axis_index("c")
    s_axis = lax.axis_index("s")

    def fire_in(i, s, in_base):
        start = in_base + i * CHUNK
        pltpu.async_copy(mask_hbm.at[pl.ds(start, CHUNK)], idx_v[s], isem[s])
        pltpu.async_copy(upd_hbm.at[pl.ds(start, CHUNK)], val_v[s], vsem[s])

    def wait_in(i, s, in_base):
        start = in_base + i * CHUNK
        pltpu.make_async_copy(mask_hbm.at[pl.ds(start, CHUNK)], idx_v[s],
                              isem[s]).wait()
        pltpu.make_async_copy(upd_hbm.at[pl.ds(start, CHUNK)], val_v[s],
                              vsem[s]).wait()

    def fire_add(s):
        pltpu.async_copy(val_v[s], win_sh.at[idx_v[s]], asem[s], add=True)

    def wait_add(s):
        pltpu.make_async_copy(val_v[s], win_sh.at[idx_v[s]], asem[s]).wait()

    def pass_body(bp, carry):
        bi = bp >> 3
        p = bp & (PASSES - 1)
        b = bi * NC + c
        lo = p * WIN
        in_base = b * N_IN + s_axis * PER_TILE

        def zfill(j, cv):
            val_v0[pl.ds(j * L, L)] = jnp.zeros((L,), jnp.float32)
            return cv

        lax.fori_loop(0, CHUNK // L, zfill, 0)

        def zcopy(z, cv):
            pltpu.sync_copy(
                val_v0,
                win_sh.at[pl.ds(s_axis * TILE_WIN + z * CHUNK, CHUNK)])
            return cv

        lax.fori_loop(0, TILE_WIN // CHUNK, zcopy, 0)
        plsc.subcore_barrier()

        fire_in(0, 0, in_base)
        fire_in(1, 1, in_base)

        def group_body(g, carry2):
            for s in range(NSETS):
                i = g * NSETS + s
                wait_in(i, s, in_base)

                def vec_body(j, cv, s=s):
                    o = j * L
                    iv = idx_v[s][pl.ds(o, L)]
                    u = iv - lo
                    inwin = plsc.bitcast(u, jnp.uint32) < jnp.uint32(WIN)
                    idx_v[s][pl.ds(o, L)] = jnp.where(inwin, u, iv & SPREAD)
                    vv = val_v[s][pl.ds(o, L)]
                    val_v[s][pl.ds(o, L)] = jnp.where(
                        inwin, vv, jnp.zeros((L,), jnp.float32))
                    return cv

                lax.fori_loop(0, CHUNK // L, vec_body, 0, unroll=4)

                fire_add(s)
                s2 = (s + 2) % NSETS

                @pl.when(i >= 2)
                def _():
                    wait_add(s2)

                @pl.when(i <= N_CHUNKS - 3)
                def _():
                    fire_in(i + 2, s2, in_base)
            return carry2

        lax.fori_loop(0, N_GROUPS, group_body, 0)
        wait_add((N_CHUNKS - 2) % NSETS)
        wait_add((N_CHUNKS - 1) % NSETS)
        plsc.subcore_barrier()

        out_start = b * N_OUT + lo + s_axis * TILE_WIN
        pltpu.sync_copy(win_sh.at[pl.ds(s_axis * TILE_WIN, TILE_WIN)],
                        out_hbm.at[pl.ds(out_start, TILE_WIN)])
        return carry

    lax.fori_loop(0, BATCHES_PER_CORE * PASSES, pass_body, 0)


_unpool = pl.kernel(
    _unpool_body,
    out_type=jax.ShapeDtypeStruct((B * N_OUT,), jnp.float32),
    mesh=plsc.VectorSubcoreMesh(core_axis_name="c", subcore_axis_name="s",
                                num_cores=NC, num_subcores=NS),
    scratch_types=(
        [pltpu.VMEM((CHUNK,), jnp.int32) for _ in range(NSETS)]
        + [pltpu.VMEM((CHUNK,), jnp.float32) for _ in range(NSETS)]
        + [pltpu.VMEM_SHARED((WIN,), jnp.float32)]
        + [pltpu.SemaphoreType.DMA] * (3 * NSETS)
    ),
)


@jax.jit
def kernel(updates, mask):
    upd = updates.reshape(-1)
    msk = mask.reshape(-1).astype(jnp.int32)
    out = _unpool(upd, msk)
    return out.reshape(B, H2, W2, C)
